# baseline (speedup 1.0000x reference)
import functools

import jax
import jax.numpy as jnp
from jax import lax
from jax.experimental import pallas as pl
from jax.experimental.pallas import tpu as pltpu
from jax.experimental.pallas import tpu_sc as plsc

_EPS = 1e-5


def _mm(a, w):
    return jnp.dot(a, w, preferred_element_type=jnp.float32,
                   precision=lax.Precision.HIGHEST)


def _topk_body(c_ref, ct_ref, idx_ref, ang_ref, *, n, rb, kp1):
    b = pl.program_id(0)
    cf = c_ref[0]
    ct = ct_ref[0]
    xf = cf[0:1, :]
    yf = cf[1:2, :]
    xr = ct[:, 0:1]
    yr = ct[:, 1:2]
    sf = xf * xf + yf * yf
    sr = xr * xr + yr * yr
    dot = xr * xf + yr * yf
    dist = jnp.maximum((-2.0 * dot + sr) + sf, 1e-12)
    cosm = dot / (jnp.sqrt(sr) * jnp.sqrt(sf))

    colid = lax.broadcasted_iota(jnp.int32, (rb, n), 1)
    lane = lax.broadcasted_iota(jnp.int32, (rb, 16), 1)
    idx_acc = jnp.zeros((rb, 16), jnp.int32)
    ang_acc = jnp.zeros((rb, 16), jnp.float32)
    d = dist
    for t in range(kp1):
        mv = jnp.min(d, axis=1, keepdims=True)
        am = jnp.min(jnp.where(d == mv, colid, n), axis=1, keepdims=True)
        sel = colid == am
        if t > 0:
            av = jnp.sum(jnp.where(sel, cosm, 0.0), axis=1, keepdims=True)
            idx_acc = jnp.where(lane == t, am + b * n, idx_acc)
            ang_acc = jnp.where(lane == t, av, ang_acc)
        d = jnp.where(sel, jnp.float32(jnp.inf), d)
    idx_ref[0] = idx_acc
    ang_ref[0] = ang_acc


def _topk(coords, kp1):
    B, _, N = coords.shape
    RB = 512
    coords_t = jnp.swapaxes(coords, 1, 2)
    body = functools.partial(_topk_body, n=N, rb=RB, kp1=kp1)
    return pl.pallas_call(
        body,
        grid=(B, N // RB),
        in_specs=[
            pl.BlockSpec((1, 2, N), lambda b, r: (b, 0, 0)),
            pl.BlockSpec((1, RB, 2), lambda b, r: (b, r, 0)),
        ],
        out_specs=[
            pl.BlockSpec((1, RB, 16), lambda b, r: (b, r, 0)),
            pl.BlockSpec((1, RB, 16), lambda b, r: (b, r, 0)),
        ],
        out_shape=[
            jax.ShapeDtypeStruct((B, N, 16), jnp.int32),
            jax.ShapeDtypeStruct((B, N, 16), jnp.float32),
        ],
    )(coords, coords_t)


def _gather_rows(table, idx3d):
    nw, per, _ = idx3d.shape
    _, C = table.shape
    info = plsc.get_sparse_core_info()
    mesh = plsc.VectorSubcoreMesh(core_axis_name="c", subcore_axis_name="s")

    @functools.partial(
        pl.kernel,
        mesh=mesh,
        out_type=jax.ShapeDtypeStruct((nw * per * 128, C), jnp.float32),
        scratch_types=[
            pltpu.VMEM((per, 128), jnp.int32),
            pltpu.VMEM((per, 128, C), jnp.float32),
            pltpu.SemaphoreType.DMA,
            pltpu.SemaphoreType.DMA,
        ],
    )
    def k(table_hbm, idx_hbm, out_hbm, idx_v, rows_v, sem, sem2):
        wid = lax.axis_index("s") * info.num_cores + lax.axis_index("c")
        pltpu.sync_copy(idx_hbm.at[wid], idx_v)
        cps = [
            pltpu.async_copy(table_hbm.at[idx_v.at[j]], rows_v.at[j], sem)
            for j in range(per)
        ]
        for cp in cps:
            cp.wait()
        for j in range(per):
            pltpu.sync_copy(
                rows_v.at[j], out_hbm.at[pl.ds((wid * per + j) * 128, 128)])

    return k(table, idx3d)


def _gather_rows2(t1, t2, idx3d):
    nw, per, _ = idx3d.shape
    _, C = t1.shape
    info = plsc.get_sparse_core_info()
    mesh = plsc.VectorSubcoreMesh(core_axis_name="c", subcore_axis_name="s")

    @functools.partial(
        pl.kernel,
        mesh=mesh,
        out_type=[
            jax.ShapeDtypeStruct((nw * per * 128, C), jnp.float32),
            jax.ShapeDtypeStruct((nw * per * 128, C), jnp.float32),
        ],
        scratch_types=[
            pltpu.VMEM((per, 128), jnp.int32),
            pltpu.VMEM((per, 128, C), jnp.float32),
            pltpu.SemaphoreType.DMA,
            pltpu.SemaphoreType.DMA,
        ],
    )
    def k(t1_hbm, t2_hbm, idx_hbm, out1_hbm, out2_hbm, idx_v, rows_v,
          sem, sem2):
        wid = lax.axis_index("s") * info.num_cores + lax.axis_index("c")
        pltpu.sync_copy(idx_hbm.at[wid], idx_v)
        cps = [
            pltpu.async_copy(t1_hbm.at[idx_v.at[j]], rows_v.at[j], sem)
            for j in range(per)
        ]
        for cp in cps:
            cp.wait()
        for j in range(per):
            pltpu.sync_copy(
                rows_v.at[j], out1_hbm.at[pl.ds((wid * per + j) * 128, 128)])
        cps2 = [
            pltpu.async_copy(t2_hbm.at[idx_v.at[j]], rows_v.at[j], sem)
            for j in range(per)
        ]
        for cp in cps2:
            cp.wait()
        for j in range(per):
            pltpu.sync_copy(
                rows_v.at[j], out2_hbm.at[pl.ds((wid * per + j) * 128, 128)])

    return k(t1, t2, idx3d)


def _stage_a_body(f_ref, nb_ref, angv_ref,
                  wfs_ref, wst_ref, b1_ref, w2s_ref, b2_ref,
                  aw1_ref, ab1_ref, aw2s_ref, ab2_ref,
                  c1d_ref, c1b_ref,
                  x1_ref, x1o_ref, fa_ref, *, n, c, k):
    F = f_ref[0]
    ANG = angv_ref[0]

    FQ = _mm(F, wfs_ref[...]) + b1_ref[...]
    z1 = [FQ, FQ, FQ]
    c1F = _mm(F, c1d_ref[...])
    s1 = jnp.zeros((1, c), jnp.float32)
    ss1 = jnp.zeros((1, c), jnp.float32)
    M1 = jnp.full((n, c), -jnp.inf, jnp.float32)
    for kk in range(k):
        D = nb_ref[0, pl.ds(kk * n, n), :] - F
        if kk < k - 1:
            z1[kk // 3] = z1[kk // 3] + _mm(D, wst_ref[kk % 3])
        z = c1F + _mm(D, c1b_ref[...])
        s1 = s1 + jnp.sum(z, axis=0, keepdims=True)
        ss1 = ss1 + jnp.sum(z * z, axis=0, keepdims=True)
        M1 = jnp.maximum(M1, z)

    cnt = jnp.float32(3 * n)
    m = (jnp.sum(z1[0], 0, keepdims=True) + jnp.sum(z1[1], 0, keepdims=True)
         + jnp.sum(z1[2], 0, keepdims=True)) / cnt
    v = (jnp.sum((z1[0] - m) ** 2, 0, keepdims=True)
         + jnp.sum((z1[1] - m) ** 2, 0, keepdims=True)
         + jnp.sum((z1[2] - m) ** 2, 0, keepdims=True)) / cnt
    inv = 1.0 / jnp.sqrt(v + _EPS)
    h = [jnp.maximum((zj - m) * inv, 0.0) for zj in z1]

    Z2 = b2_ref[...]
    for t in range(3):
        Z2 = Z2 + _mm(h[t], w2s_ref[t])
    m2 = jnp.sum(Z2, 0, keepdims=True) / n
    v2 = jnp.sum((Z2 - m2) ** 2, 0, keepdims=True) / n
    X1 = jnp.maximum((Z2 - m2) / jnp.sqrt(v2 + _EPS), 0.0)
    x1_ref[0] = X1

    aw1 = aw1_ref[...]
    za = []
    for j in range(3):
        acc = ab1_ref[...]
        for t in range(3):
            col = ANG[:, 1 + 3 * j + t: 2 + 3 * j + t]
            acc = acc + col * aw1[t:t + 1, :]
        za.append(acc)
    ma = (jnp.sum(za[0], 0, keepdims=True) + jnp.sum(za[1], 0, keepdims=True)
          + jnp.sum(za[2], 0, keepdims=True)) / cnt
    va = (jnp.sum((za[0] - ma) ** 2, 0, keepdims=True)
          + jnp.sum((za[1] - ma) ** 2, 0, keepdims=True)
          + jnp.sum((za[2] - ma) ** 2, 0, keepdims=True)) / cnt
    inva = 1.0 / jnp.sqrt(va + _EPS)
    ha = [jnp.maximum((zj - ma) * inva, 0.0) for zj in za]
    FA = ab2_ref[...]
    for t in range(3):
        FA = FA + _mm(ha[t], aw2s_ref[t])
    mfa = jnp.sum(FA, 0, keepdims=True) / n
    vfa = jnp.sum((FA - mfa) ** 2, 0, keepdims=True) / n
    fa_ref[0] = jnp.maximum((FA - mfa) / jnp.sqrt(vfa + _EPS), 0.0)

    cnt1 = jnp.float32(k * n)
    m1 = s1 / cnt1
    v1 = ss1 / cnt1 - m1 * m1
    zn = (M1 - m1) / jnp.sqrt(v1 + _EPS)
    x1o_ref[0] = jnp.where(zn >= 0, zn, 0.2 * zn)


def _stage_a(F, NB, ANGV, wfs, wst, b1, w2s, b2, aw1, ab1, aw2s, ab2,
             c1d, c1b):
    B, N, C = F.shape
    K = NB.shape[1] // N
    body = functools.partial(_stage_a_body, n=N, c=C, k=K)
    full = lambda s: pl.BlockSpec(s, lambda b: (0,) * len(s))
    return pl.pallas_call(
        body,
        grid=(B,),
        in_specs=[
            pl.BlockSpec((1, N, C), lambda b: (b, 0, 0)),
            pl.BlockSpec((1, K * N, C), lambda b: (b, 0, 0)),
            pl.BlockSpec((1, N, 16), lambda b: (b, 0, 0)),
            full((C, C)), full((3, C, C)), full((1, C)),
            full((3, C, C)), full((1, C)),
            full((3, C)), full((1, C)), full((3, C, C)), full((1, C)),
            full((C, C)), full((C, C)),
        ],
        out_specs=[
            pl.BlockSpec((1, N, C), lambda b: (b, 0, 0)),
            pl.BlockSpec((1, N, C), lambda b: (b, 0, 0)),
            pl.BlockSpec((1, N, C), lambda b: (b, 0, 0)),
        ],
        out_shape=[
            jax.ShapeDtypeStruct((B, N, C), jnp.float32),
            jax.ShapeDtypeStruct((B, N, C), jnp.float32),
            jax.ShapeDtypeStruct((B, N, C), jnp.float32),
        ],
    )(F, NB, ANGV, wfs, wst, b1, w2s, b2, aw1, ab1, aw2s, ab2, c1d, c1b)


def _stage_b_body(f_ref, x1_ref, x1o_ref, fa_ref, nb1_ref, nbo_ref,
                  wfs2_ref, wst2_ref, b21_ref, w22s_ref, b22_ref,
                  c2d_ref, c2b_ref,
                  w3a_ref, w3b_ref, w3c_ref,
                  w3oa_ref, w3ob_ref, w3oc_ref,
                  out_ref, *, n, c, k):
    F = f_ref[0]
    X1 = x1_ref[0]
    X1o = x1o_ref[0]
    FA = fa_ref[0]
    c2 = 2 * c

    FQ2 = _mm(X1, wfs2_ref[...]) + b21_ref[...]
    z2 = [FQ2, FQ2, FQ2]
    c2F = _mm(X1o, c2d_ref[...])
    s2 = jnp.zeros((1, c2), jnp.float32)
    ss2 = jnp.zeros((1, c2), jnp.float32)
    M2 = jnp.full((n, c2), -jnp.inf, jnp.float32)
    for kk in range(k):
        D1 = nb1_ref[0, pl.ds(kk * n, n), :] - X1
        if kk < k - 1:
            z2[kk // 3] = z2[kk // 3] + _mm(D1, wst2_ref[kk % 3])
        Do = nbo_ref[0, pl.ds(kk * n, n), :] - X1o
        z = c2F + _mm(Do, c2b_ref[...])
        s2 = s2 + jnp.sum(z, axis=0, keepdims=True)
        ss2 = ss2 + jnp.sum(z * z, axis=0, keepdims=True)
        M2 = jnp.maximum(M2, z)
    cnt = jnp.float32(3 * n)
    m = (jnp.sum(z2[0], 0, keepdims=True) + jnp.sum(z2[1], 0, keepdims=True)
         + jnp.sum(z2[2], 0, keepdims=True)) / cnt
    v = (jnp.sum((z2[0] - m) ** 2, 0, keepdims=True)
         + jnp.sum((z2[1] - m) ** 2, 0, keepdims=True)
         + jnp.sum((z2[2] - m) ** 2, 0, keepdims=True)) / cnt
    inv = 1.0 / jnp.sqrt(v + _EPS)
    h = [jnp.maximum((zj - m) * inv, 0.0) for zj in z2]
    Z22 = b22_ref[...]
    for t in range(3):
        Z22 = Z22 + _mm(h[t], w22s_ref[t])
    m2 = jnp.sum(Z22, 0, keepdims=True) / n
    v2 = jnp.sum((Z22 - m2) ** 2, 0, keepdims=True) / n
    X2 = jnp.maximum((Z22 - m2) / jnp.sqrt(v2 + _EPS), 0.0)

    cnt2 = jnp.float32(k * n)
    m2o = s2 / cnt2
    v2o = ss2 / cnt2 - m2o * m2o
    zn = (M2 - m2o) / jnp.sqrt(v2o + _EPS)
    X2o = jnp.where(zn >= 0, zn, 0.2 * zn)

    Z3 = _mm(F, w3a_ref[...]) + _mm(X1 + FA, w3b_ref[...]) \
        + _mm(X2 + FA, w3c_ref[...])
    m3 = jnp.sum(Z3, 0, keepdims=True) / n
    v3 = jnp.sum((Z3 - m3) ** 2, 0, keepdims=True) / n
    z3n = (Z3 - m3) / jnp.sqrt(v3 + _EPS)
    z3n = jnp.where(z3n >= 0, z3n, 0.2 * z3n)

    Z3o = _mm(F, w3oa_ref[...]) + _mm(X1o, w3ob_ref[...]) \
        + _mm(X2o, w3oc_ref[...])
    m3o = jnp.sum(Z3o, 0, keepdims=True) / n
    v3o = jnp.sum((Z3o - m3o) ** 2, 0, keepdims=True) / n
    z3on = (Z3o - m3o) / jnp.sqrt(v3o + _EPS)
    z3on = jnp.where(z3on >= 0, z3on, 0.2 * z3on)

    out_ref[0] = z3n + z3on


def _stage_b(F, X1, X1o, FA, NB1, NBo,
             wfs2, wst2, b21, w22s, b22, c2d, c2b,
             w3a, w3b, w3c, w3oa, w3ob, w3oc):
    B, N, C = F.shape
    K = NB1.shape[1] // N
    body = functools.partial(_stage_b_body, n=N, c=C, k=K)
    full = lambda s: pl.BlockSpec(s, lambda b: (0,) * len(s))
    bspec = lambda ch: pl.BlockSpec((1, N, ch), lambda b: (b, 0, 0))
    return pl.pallas_call(
        body,
        grid=(B,),
        in_specs=[
            bspec(C), bspec(C), bspec(C), bspec(C),
            pl.BlockSpec((1, K * N, C), lambda b: (b, 0, 0)),
            pl.BlockSpec((1, K * N, C), lambda b: (b, 0, 0)),
            full((C, C)), full((3, C, C)), full((1, C)),
            full((3, C, C)), full((1, C)),
            full((C, 2 * C)), full((C, 2 * C)),
            full((C, C)), full((C, C)), full((C, C)),
            full((C, C)), full((C, C)), full((2 * C, C)),
        ],
        out_specs=[pl.BlockSpec((1, N, C), lambda b: (b, 0, 0))],
        out_shape=[jax.ShapeDtypeStruct((B, N, C), jnp.float32)],
    )(F, X1, X1o, FA, NB1, NBo,
      wfs2, wst2, b21, w22s, b22, c2d, c2b,
      w3a, w3b, w3c, w3oa, w3ob, w3oc)


def kernel(coords, features, conv1_w, conv2_w, conv3_w, conv3_old_w,
           an1_c1_w, an1_c1_b, an1_c2_w, an1_c2_b,
           an2_c1_w, an2_c1_b, an2_c2_w, an2_c2_b,
           ang_c1_w, ang_c1_b, ang_c2_w, ang_c2_b):
    B, C, N = features.shape
    K = 10
    NW = 32

    idx16, angv = _topk(coords, K + 1)
    idxT = jnp.swapaxes(idx16[:, :, 1:K + 1], 1, 2)
    idx3d = idxT.reshape(NW, B * K * N // (NW * 128), 128)

    F = jnp.swapaxes(features, 1, 2)
    FT = F.reshape(B * N, C)
    NBF = _gather_rows(FT, idx3d).reshape(B, K * N, C)

    wn1 = jnp.transpose(an1_c1_w[:, C:, 0, :], (2, 1, 0))
    wfs1 = jnp.sum(jnp.transpose(an1_c1_w[:, :C, 0, :], (2, 1, 0)), axis=0)
    wst1 = wn1
    w2s1 = jnp.transpose(an1_c2_w[:, :, 0, :], (2, 1, 0))
    aw1 = jnp.transpose(ang_c1_w[:, 0, 0, :], (1, 0))
    aw2s = jnp.transpose(ang_c2_w[:, :, 0, :], (2, 1, 0))
    c1a = jnp.transpose(conv1_w[:, :C, 0, 0])
    c1b = jnp.transpose(conv1_w[:, C:, 0, 0])
    c1d = c1a

    X1, X1o, FA = _stage_a(F, NBF, angv,
                           wfs1, wst1, an1_c1_b.reshape(1, C), w2s1,
                           an1_c2_b.reshape(1, C), aw1, ang_c1_b.reshape(1, C),
                           aw2s, ang_c2_b.reshape(1, C), c1d, c1b)

    NB1, NBo = _gather_rows2(X1.reshape(B * N, C), X1o.reshape(B * N, C), idx3d)
    NB1 = NB1.reshape(B, K * N, C)
    NBo = NBo.reshape(B, K * N, C)

    wn2 = jnp.transpose(an2_c1_w[:, C:, 0, :], (2, 1, 0))
    wfs2 = jnp.sum(jnp.transpose(an2_c1_w[:, :C, 0, :], (2, 1, 0)), axis=0)
    wst2 = wn2
    w22s = jnp.transpose(an2_c2_w[:, :, 0, :], (2, 1, 0))
    c2a = jnp.transpose(conv2_w[:, :C, 0, 0])
    c2b = jnp.transpose(conv2_w[:, C:, 0, 0])
    c2d = c2a
    w3a = jnp.transpose(conv3_w[:, :C, 0, 0])
    w3b = jnp.transpose(conv3_w[:, C:2 * C, 0, 0])
    w3c = jnp.transpose(conv3_w[:, 2 * C:, 0, 0])
    w3oa = jnp.transpose(conv3_old_w[:, :C, 0, 0])
    w3ob = jnp.transpose(conv3_old_w[:, C:2 * C, 0, 0])
    w3oc = jnp.transpose(conv3_old_w[:, 2 * C:, 0, 0])

    out = _stage_b(F, X1, X1o, FA, NB1, NBo,
                   wfs2, wst2, an2_c1_b.reshape(1, C), w22s,
                   an2_c2_b.reshape(1, C), c2d, c2b,
                   w3a, w3b, w3c, w3oa, w3ob, w3oc)[0]
    return jnp.swapaxes(out, 1, 2)


# speedup vs baseline: 1.0687x; 1.0687x over previous
import functools

import jax
import jax.numpy as jnp
from jax import lax
from jax.experimental import pallas as pl
from jax.experimental.pallas import tpu as pltpu
from jax.experimental.pallas import tpu_sc as plsc

_EPS = 1e-5


def _mm(a, w):
    return jnp.dot(a, w, preferred_element_type=jnp.float32,
                   precision=lax.Precision.HIGHEST)


def _topk_body(c_ref, ct_ref, feat_ref, idx_ref, ang_ref, f_ref,
               *, n, rb, kp1):
    b = pl.program_id(0)
    cf = c_ref[0]
    ct = ct_ref[0]
    xf = cf[0:1, :]
    yf = cf[1:2, :]
    xr = ct[:, 0:1]
    yr = ct[:, 1:2]
    sf = xf * xf + yf * yf
    sr = xr * xr + yr * yr
    dot = xr * xf + yr * yf
    dist = jnp.maximum((-2.0 * dot + sr) + sf, 1e-12)
    cosm = dot / (jnp.sqrt(sr) * jnp.sqrt(sf))

    colf = lax.broadcasted_iota(jnp.int32, (rb, n), 1).astype(jnp.float32)
    lane = lax.broadcasted_iota(jnp.int32, (rb, 16), 1)
    idx_acc = jnp.zeros((rb, 16), jnp.int32)
    ang_acc = jnp.zeros((rb, 16), jnp.float32)
    d = dist
    for t in range(kp1):
        mv = jnp.min(d, axis=1, keepdims=True)
        amf = jnp.min(jnp.where(d == mv, colf, jnp.float32(n)),
                      axis=1, keepdims=True)
        sel = colf == amf
        if t > 0:
            av = jnp.sum(jnp.where(sel, cosm, 0.0), axis=1, keepdims=True)
            ami = amf.astype(jnp.int32)
            idx_acc = jnp.where(lane == t, ami + b * n, idx_acc)
            ang_acc = jnp.where(lane == t, av, ang_acc)
        d = jnp.where(sel, jnp.float32(jnp.inf), d)
    idx_ref[0] = jnp.swapaxes(idx_acc, 0, 1)[1:kp1, :]
    ang_ref[0] = ang_acc
    f_ref[0] = jnp.swapaxes(feat_ref[0], 0, 1)


def _topk(coords, features, kp1):
    B, C, N = features.shape
    RB = 512
    coords_t = jnp.swapaxes(coords, 1, 2)
    body = functools.partial(_topk_body, n=N, rb=RB, kp1=kp1)
    return pl.pallas_call(
        body,
        grid=(B, N // RB),
        in_specs=[
            pl.BlockSpec((1, 2, N), lambda b, r: (b, 0, 0)),
            pl.BlockSpec((1, RB, 2), lambda b, r: (b, r, 0)),
            pl.BlockSpec((1, C, RB), lambda b, r: (b, 0, r)),
        ],
        out_specs=[
            pl.BlockSpec((1, kp1 - 1, RB), lambda b, r: (b, 0, r)),
            pl.BlockSpec((1, RB, 16), lambda b, r: (b, r, 0)),
            pl.BlockSpec((1, RB, C), lambda b, r: (b, r, 0)),
        ],
        out_shape=[
            jax.ShapeDtypeStruct((B, kp1 - 1, N), jnp.int32),
            jax.ShapeDtypeStruct((B, N, 16), jnp.float32),
            jax.ShapeDtypeStruct((B, N, C), jnp.float32),
        ],
    )(coords, coords_t, features)


def _gather_rows(table, idx3d):
    nw, per, _ = idx3d.shape
    _, C = table.shape
    info = plsc.get_sparse_core_info()
    mesh = plsc.VectorSubcoreMesh(core_axis_name="c", subcore_axis_name="s")

    @functools.partial(
        pl.kernel,
        mesh=mesh,
        out_type=jax.ShapeDtypeStruct((nw * per * 128, C), jnp.float32),
        scratch_types=[
            pltpu.VMEM((per, 128), jnp.int32),
            pltpu.VMEM((per, 128, C), jnp.float32),
            pltpu.SemaphoreType.DMA,
            pltpu.SemaphoreType.DMA,
        ],
    )
    def k(table_hbm, idx_hbm, out_hbm, idx_v, rows_v, sem, sem2):
        wid = lax.axis_index("s") * info.num_cores + lax.axis_index("c")
        pltpu.sync_copy(idx_hbm.at[wid], idx_v)
        cps = [
            pltpu.async_copy(table_hbm.at[idx_v.at[j]], rows_v.at[j], sem)
            for j in range(per)
        ]
        for cp in cps:
            cp.wait()
        for j in range(per):
            pltpu.sync_copy(
                rows_v.at[j], out_hbm.at[pl.ds((wid * per + j) * 128, 128)])

    return k(table, idx3d)


def _gather_rows2(t1, t2, idx3d):
    nw, per, _ = idx3d.shape
    _, C = t1.shape
    info = plsc.get_sparse_core_info()
    mesh = plsc.VectorSubcoreMesh(core_axis_name="c", subcore_axis_name="s")

    @functools.partial(
        pl.kernel,
        mesh=mesh,
        out_type=[
            jax.ShapeDtypeStruct((nw * per * 128, C), jnp.float32),
            jax.ShapeDtypeStruct((nw * per * 128, C), jnp.float32),
        ],
        scratch_types=[
            pltpu.VMEM((per, 128), jnp.int32),
            pltpu.VMEM((per, 128, C), jnp.float32),
            pltpu.SemaphoreType.DMA,
            pltpu.SemaphoreType.DMA,
        ],
    )
    def k(t1_hbm, t2_hbm, idx_hbm, out1_hbm, out2_hbm, idx_v, rows_v,
          sem, sem2):
        wid = lax.axis_index("s") * info.num_cores + lax.axis_index("c")
        pltpu.sync_copy(idx_hbm.at[wid], idx_v)
        cps = [
            pltpu.async_copy(t1_hbm.at[idx_v.at[j]], rows_v.at[j], sem)
            for j in range(per)
        ]
        for cp in cps:
            cp.wait()
        for j in range(per):
            pltpu.sync_copy(
                rows_v.at[j], out1_hbm.at[pl.ds((wid * per + j) * 128, 128)])
        cps2 = [
            pltpu.async_copy(t2_hbm.at[idx_v.at[j]], rows_v.at[j], sem)
            for j in range(per)
        ]
        for cp in cps2:
            cp.wait()
        for j in range(per):
            pltpu.sync_copy(
                rows_v.at[j], out2_hbm.at[pl.ds((wid * per + j) * 128, 128)])

    return k(t1, t2, idx3d)


def _stage_a_body(f_ref, nb_ref, angv_ref,
                  wfs_ref, wst_ref, b1_ref, w2s_ref, b2_ref,
                  aw1_ref, ab1_ref, aw2s_ref, ab2_ref,
                  c1d_ref, c1b_ref,
                  x1_ref, x1o_ref, fa_ref, *, n, c, k):
    F = f_ref[0]
    ANG = angv_ref[0]

    FQ = _mm(F, wfs_ref[...]) + b1_ref[...]
    z1 = [FQ, FQ, FQ]
    c1F = _mm(F, c1d_ref[...])
    s1 = jnp.zeros((1, c), jnp.float32)
    ss1 = jnp.zeros((1, c), jnp.float32)
    M1 = jnp.full((n, c), -jnp.inf, jnp.float32)
    for kk in range(k):
        D = nb_ref[0, pl.ds(kk * n, n), :] - F
        if kk < k - 1:
            z1[kk // 3] = z1[kk // 3] + _mm(D, wst_ref[kk % 3])
        z = c1F + _mm(D, c1b_ref[...])
        s1 = s1 + jnp.sum(z, axis=0, keepdims=True)
        ss1 = ss1 + jnp.sum(z * z, axis=0, keepdims=True)
        M1 = jnp.maximum(M1, z)

    cnt = jnp.float32(3 * n)
    m = (jnp.sum(z1[0], 0, keepdims=True) + jnp.sum(z1[1], 0, keepdims=True)
         + jnp.sum(z1[2], 0, keepdims=True)) / cnt
    v = (jnp.sum((z1[0] - m) ** 2, 0, keepdims=True)
         + jnp.sum((z1[1] - m) ** 2, 0, keepdims=True)
         + jnp.sum((z1[2] - m) ** 2, 0, keepdims=True)) / cnt
    inv = 1.0 / jnp.sqrt(v + _EPS)
    h = [jnp.maximum((zj - m) * inv, 0.0) for zj in z1]

    Z2 = b2_ref[...]
    for t in range(3):
        Z2 = Z2 + _mm(h[t], w2s_ref[t])
    m2 = jnp.sum(Z2, 0, keepdims=True) / n
    v2 = jnp.sum((Z2 - m2) ** 2, 0, keepdims=True) / n
    X1 = jnp.maximum((Z2 - m2) / jnp.sqrt(v2 + _EPS), 0.0)
    x1_ref[0] = X1

    aw1 = aw1_ref[...]
    za = []
    for j in range(3):
        acc = ab1_ref[...]
        for t in range(3):
            col = ANG[:, 1 + 3 * j + t: 2 + 3 * j + t]
            acc = acc + col * aw1[t:t + 1, :]
        za.append(acc)
    ma = (jnp.sum(za[0], 0, keepdims=True) + jnp.sum(za[1], 0, keepdims=True)
          + jnp.sum(za[2], 0, keepdims=True)) / cnt
    va = (jnp.sum((za[0] - ma) ** 2, 0, keepdims=True)
          + jnp.sum((za[1] - ma) ** 2, 0, keepdims=True)
          + jnp.sum((za[2] - ma) ** 2, 0, keepdims=True)) / cnt
    inva = 1.0 / jnp.sqrt(va + _EPS)
    ha = [jnp.maximum((zj - ma) * inva, 0.0) for zj in za]
    FA = ab2_ref[...]
    for t in range(3):
        FA = FA + _mm(ha[t], aw2s_ref[t])
    mfa = jnp.sum(FA, 0, keepdims=True) / n
    vfa = jnp.sum((FA - mfa) ** 2, 0, keepdims=True) / n
    fa_ref[0] = jnp.maximum((FA - mfa) / jnp.sqrt(vfa + _EPS), 0.0)

    cnt1 = jnp.float32(k * n)
    m1 = s1 / cnt1
    v1 = ss1 / cnt1 - m1 * m1
    zn = (M1 - m1) / jnp.sqrt(v1 + _EPS)
    x1o_ref[0] = jnp.where(zn >= 0, zn, 0.2 * zn)


def _stage_a(F, NB, ANGV, wfs, wst, b1, w2s, b2, aw1, ab1, aw2s, ab2,
             c1d, c1b):
    B, N, C = F.shape
    K = NB.shape[1] // N
    body = functools.partial(_stage_a_body, n=N, c=C, k=K)
    full = lambda s: pl.BlockSpec(s, lambda b: (0,) * len(s))
    return pl.pallas_call(
        body,
        grid=(B,),
        in_specs=[
            pl.BlockSpec((1, N, C), lambda b: (b, 0, 0)),
            pl.BlockSpec((1, K * N, C), lambda b: (b, 0, 0)),
            pl.BlockSpec((1, N, 16), lambda b: (b, 0, 0)),
            full((C, C)), full((3, C, C)), full((1, C)),
            full((3, C, C)), full((1, C)),
            full((3, C)), full((1, C)), full((3, C, C)), full((1, C)),
            full((C, C)), full((C, C)),
        ],
        out_specs=[
            pl.BlockSpec((1, N, C), lambda b: (b, 0, 0)),
            pl.BlockSpec((1, N, C), lambda b: (b, 0, 0)),
            pl.BlockSpec((1, N, C), lambda b: (b, 0, 0)),
        ],
        out_shape=[
            jax.ShapeDtypeStruct((B, N, C), jnp.float32),
            jax.ShapeDtypeStruct((B, N, C), jnp.float32),
            jax.ShapeDtypeStruct((B, N, C), jnp.float32),
        ],
    )(F, NB, ANGV, wfs, wst, b1, w2s, b2, aw1, ab1, aw2s, ab2, c1d, c1b)


def _stage_b_body(f_ref, x1_ref, x1o_ref, fa_ref, nb1_ref, nbo_ref,
                  wfs2_ref, wst2_ref, b21_ref, w22s_ref, b22_ref,
                  c2d_ref, c2b_ref,
                  w3a_ref, w3b_ref, w3c_ref,
                  w3oa_ref, w3ob_ref, w3oc_ref,
                  out_ref, *, n, c, k):
    F = f_ref[0]
    X1 = x1_ref[0]
    X1o = x1o_ref[0]
    FA = fa_ref[0]
    c2 = 2 * c

    FQ2 = _mm(X1, wfs2_ref[...]) + b21_ref[...]
    z2 = [FQ2, FQ2, FQ2]
    c2F = _mm(X1o, c2d_ref[...])
    s2 = jnp.zeros((1, c2), jnp.float32)
    ss2 = jnp.zeros((1, c2), jnp.float32)
    M2 = jnp.full((n, c2), -jnp.inf, jnp.float32)
    for kk in range(k):
        D1 = nb1_ref[0, pl.ds(kk * n, n), :] - X1
        if kk < k - 1:
            z2[kk // 3] = z2[kk // 3] + _mm(D1, wst2_ref[kk % 3])
        Do = nbo_ref[0, pl.ds(kk * n, n), :] - X1o
        z = c2F + _mm(Do, c2b_ref[...])
        s2 = s2 + jnp.sum(z, axis=0, keepdims=True)
        ss2 = ss2 + jnp.sum(z * z, axis=0, keepdims=True)
        M2 = jnp.maximum(M2, z)
    cnt = jnp.float32(3 * n)
    m = (jnp.sum(z2[0], 0, keepdims=True) + jnp.sum(z2[1], 0, keepdims=True)
         + jnp.sum(z2[2], 0, keepdims=True)) / cnt
    v = (jnp.sum((z2[0] - m) ** 2, 0, keepdims=True)
         + jnp.sum((z2[1] - m) ** 2, 0, keepdims=True)
         + jnp.sum((z2[2] - m) ** 2, 0, keepdims=True)) / cnt
    inv = 1.0 / jnp.sqrt(v + _EPS)
    h = [jnp.maximum((zj - m) * inv, 0.0) for zj in z2]
    Z22 = b22_ref[...]
    for t in range(3):
        Z22 = Z22 + _mm(h[t], w22s_ref[t])
    m2 = jnp.sum(Z22, 0, keepdims=True) / n
    v2 = jnp.sum((Z22 - m2) ** 2, 0, keepdims=True) / n
    X2 = jnp.maximum((Z22 - m2) / jnp.sqrt(v2 + _EPS), 0.0)

    cnt2 = jnp.float32(k * n)
    m2o = s2 / cnt2
    v2o = ss2 / cnt2 - m2o * m2o
    zn = (M2 - m2o) / jnp.sqrt(v2o + _EPS)
    X2o = jnp.where(zn >= 0, zn, 0.2 * zn)

    Z3 = _mm(F, w3a_ref[...]) + _mm(X1 + FA, w3b_ref[...]) \
        + _mm(X2 + FA, w3c_ref[...])
    m3 = jnp.sum(Z3, 0, keepdims=True) / n
    v3 = jnp.sum((Z3 - m3) ** 2, 0, keepdims=True) / n
    z3n = (Z3 - m3) / jnp.sqrt(v3 + _EPS)
    z3n = jnp.where(z3n >= 0, z3n, 0.2 * z3n)

    Z3o = _mm(F, w3oa_ref[...]) + _mm(X1o, w3ob_ref[...]) \
        + _mm(X2o, w3oc_ref[...])
    m3o = jnp.sum(Z3o, 0, keepdims=True) / n
    v3o = jnp.sum((Z3o - m3o) ** 2, 0, keepdims=True) / n
    z3on = (Z3o - m3o) / jnp.sqrt(v3o + _EPS)
    z3on = jnp.where(z3on >= 0, z3on, 0.2 * z3on)

    out_ref[0] = jnp.swapaxes(z3n + z3on, 0, 1)


def _stage_b(F, X1, X1o, FA, NB1, NBo,
             wfs2, wst2, b21, w22s, b22, c2d, c2b,
             w3a, w3b, w3c, w3oa, w3ob, w3oc):
    B, N, C = F.shape
    K = NB1.shape[1] // N
    body = functools.partial(_stage_b_body, n=N, c=C, k=K)
    full = lambda s: pl.BlockSpec(s, lambda b: (0,) * len(s))
    bspec = lambda ch: pl.BlockSpec((1, N, ch), lambda b: (b, 0, 0))
    return pl.pallas_call(
        body,
        grid=(B,),
        in_specs=[
            bspec(C), bspec(C), bspec(C), bspec(C),
            pl.BlockSpec((1, K * N, C), lambda b: (b, 0, 0)),
            pl.BlockSpec((1, K * N, C), lambda b: (b, 0, 0)),
            full((C, C)), full((3, C, C)), full((1, C)),
            full((3, C, C)), full((1, C)),
            full((C, 2 * C)), full((C, 2 * C)),
            full((C, C)), full((C, C)), full((C, C)),
            full((C, C)), full((C, C)), full((2 * C, C)),
        ],
        out_specs=[pl.BlockSpec((1, C, N), lambda b: (b, 0, 0))],
        out_shape=[jax.ShapeDtypeStruct((B, C, N), jnp.float32)],
    )(F, X1, X1o, FA, NB1, NBo,
      wfs2, wst2, b21, w22s, b22, c2d, c2b,
      w3a, w3b, w3c, w3oa, w3ob, w3oc)


def kernel(coords, features, conv1_w, conv2_w, conv3_w, conv3_old_w,
           an1_c1_w, an1_c1_b, an1_c2_w, an1_c2_b,
           an2_c1_w, an2_c1_b, an2_c2_w, an2_c2_b,
           ang_c1_w, ang_c1_b, ang_c2_w, ang_c2_b):
    B, C, N = features.shape
    K = 10
    NW = 32

    idxT, angv, F = _topk(coords, features, K + 1)
    idx3d = idxT.reshape(NW, B * K * N // (NW * 128), 128)
    NBF = _gather_rows(F.reshape(B * N, C), idx3d).reshape(B, K * N, C)

    wn1 = jnp.transpose(an1_c1_w[:, C:, 0, :], (2, 1, 0))
    wfs1 = jnp.sum(jnp.transpose(an1_c1_w[:, :C, 0, :], (2, 1, 0)), axis=0)
    wst1 = wn1
    w2s1 = jnp.transpose(an1_c2_w[:, :, 0, :], (2, 1, 0))
    aw1 = jnp.transpose(ang_c1_w[:, 0, 0, :], (1, 0))
    aw2s = jnp.transpose(ang_c2_w[:, :, 0, :], (2, 1, 0))
    c1a = jnp.transpose(conv1_w[:, :C, 0, 0])
    c1b = jnp.transpose(conv1_w[:, C:, 0, 0])
    c1d = c1a

    X1, X1o, FA = _stage_a(F, NBF, angv,
                           wfs1, wst1, an1_c1_b.reshape(1, C), w2s1,
                           an1_c2_b.reshape(1, C), aw1, ang_c1_b.reshape(1, C),
                           aw2s, ang_c2_b.reshape(1, C), c1d, c1b)

    NB1, NBo = _gather_rows2(X1.reshape(B * N, C), X1o.reshape(B * N, C), idx3d)
    NB1 = NB1.reshape(B, K * N, C)
    NBo = NBo.reshape(B, K * N, C)

    wn2 = jnp.transpose(an2_c1_w[:, C:, 0, :], (2, 1, 0))
    wfs2 = jnp.sum(jnp.transpose(an2_c1_w[:, :C, 0, :], (2, 1, 0)), axis=0)
    wst2 = wn2
    w22s = jnp.transpose(an2_c2_w[:, :, 0, :], (2, 1, 0))
    c2a = jnp.transpose(conv2_w[:, :C, 0, 0])
    c2b = jnp.transpose(conv2_w[:, C:, 0, 0])
    c2d = c2a
    w3a = jnp.transpose(conv3_w[:, :C, 0, 0])
    w3b = jnp.transpose(conv3_w[:, C:2 * C, 0, 0])
    w3c = jnp.transpose(conv3_w[:, 2 * C:, 0, 0])
    w3oa = jnp.transpose(conv3_old_w[:, :C, 0, 0])
    w3ob = jnp.transpose(conv3_old_w[:, C:2 * C, 0, 0])
    w3oc = jnp.transpose(conv3_old_w[:, 2 * C:, 0, 0])

    out = _stage_b(F, X1, X1o, FA, NB1, NBo,
                   wfs2, wst2, an2_c1_b.reshape(1, C), w22s,
                   an2_c2_b.reshape(1, C), c2d, c2b,
                   w3a, w3b, w3c, w3oa, w3ob, w3oc)[0]
    return out
